# Initial kernel scaffold; baseline (speedup 1.0000x reference)
import jax
import jax.numpy as jnp
from jax.experimental import pallas as pl


def kernel(input, table, W):
    raise NotImplementedError("write your pallas kernel here")


# speedup vs baseline: 2.4349x; 2.4349x over previous
import functools

import jax
import jax.numpy as jnp
from jax import lax
from jax.experimental import pallas as pl
from jax.experimental.pallas import tpu as pltpu
from jax.experimental.pallas import tpu_sc as plsc

BOTTLENECK = 32
EMB = 128
NC, NS = 2, 16
NW = NC * NS
ROW_BLK = 1024
CH = 128


def _proj_body(t_ref, w_ref, o_ref):
    o_ref[...] = jnp.dot(t_ref[...], w_ref[...],
                         preferred_element_type=jnp.float32)


def _project(table_pad, W):
    rows = table_pad.shape[0]
    return pl.pallas_call(
        _proj_body,
        grid=(rows // ROW_BLK,),
        in_specs=[
            pl.BlockSpec((ROW_BLK, BOTTLENECK), lambda i: (i, 0)),
            pl.BlockSpec((BOTTLENECK, EMB), lambda i: (0, 0)),
        ],
        out_specs=pl.BlockSpec((ROW_BLK, EMB), lambda i: (i, 0)),
        out_shape=jax.ShapeDtypeStruct((rows, EMB), jnp.float32),
    )(table_pad, W)


def _make_gather(B):
    b_per_w = B // NW
    n_chunks = b_per_w // CH
    mesh = plsc.VectorSubcoreMesh(core_axis_name="c", subcore_axis_name="s")

    @functools.partial(
        pl.kernel,
        out_type=jax.ShapeDtypeStruct((B, EMB), jnp.float32),
        mesh=mesh,
        scratch_types=[
            pltpu.VMEM((n_chunks, CH), jnp.int32),
            pltpu.VMEM((CH, EMB), jnp.float32),
            pltpu.SemaphoreType.DMA,
        ],
    )
    def gather_kernel(idx_hbm, proj_hbm, out_hbm, idx_v, rows_v, sem):
        wid = lax.axis_index("s") * NC + lax.axis_index("c")
        base = wid * b_per_w
        pltpu.sync_copy(idx_hbm.at[wid], idx_v)

        def body(j, carry):
            pltpu.async_copy(proj_hbm.at[idx_v.at[j]], rows_v, sem).wait()
            pltpu.sync_copy(rows_v, out_hbm.at[pl.ds(base + j * CH, CH)])
            return carry

        lax.fori_loop(0, n_chunks, body, 0, unroll=False)

    return gather_kernel


def kernel(input, table, W):
    B = input.shape[0] * input.shape[1]
    rows_pad = ((table.shape[0] + ROW_BLK - 1) // ROW_BLK) * ROW_BLK
    table_pad = jnp.pad(table, ((0, rows_pad - table.shape[0]), (0, 0)))
    proj = _project(table_pad, W)

    idx = input.reshape(NW, B // NW // CH, CH).astype(jnp.int32)
    out = _make_gather(B)(idx, proj)
    return out.reshape(input.shape[0], input.shape[1], EMB)


# speedup vs baseline: 2.7169x; 1.1158x over previous
import functools

import jax
import jax.numpy as jnp
from jax import lax
from jax.experimental import pallas as pl
from jax.experimental.pallas import tpu as pltpu
from jax.experimental.pallas import tpu_sc as plsc

BOTTLENECK = 32
EMB = 128
NC, NS = 2, 16
NW = NC * NS
ROW_BLK = 1024
CH = 128


def _proj_body(t_ref, w_ref, o_ref):
    o_ref[...] = jnp.dot(t_ref[...], w_ref[...],
                         preferred_element_type=jnp.float32)


def _project(table_pad, W):
    rows = table_pad.shape[0]
    return pl.pallas_call(
        _proj_body,
        grid=(rows // ROW_BLK,),
        in_specs=[
            pl.BlockSpec((ROW_BLK, BOTTLENECK), lambda i: (i, 0)),
            pl.BlockSpec((BOTTLENECK, EMB), lambda i: (0, 0)),
        ],
        out_specs=pl.BlockSpec((ROW_BLK, EMB), lambda i: (i, 0)),
        out_shape=jax.ShapeDtypeStruct((rows, EMB), jnp.float32),
    )(table_pad, W)


def _make_gather(B):
    b_per_w = B // NW
    n_chunks = b_per_w // CH
    mesh = plsc.VectorSubcoreMesh(core_axis_name="c", subcore_axis_name="s")

    @functools.partial(
        pl.kernel,
        out_type=jax.ShapeDtypeStruct((B, EMB), jnp.float32),
        mesh=mesh,
        scratch_types=[
            pltpu.VMEM((n_chunks, CH), jnp.int32),
            pltpu.VMEM((CH, EMB), jnp.float32),
            pltpu.VMEM((CH, EMB), jnp.float32),
            pltpu.SemaphoreType.DMA,
            pltpu.SemaphoreType.DMA,
        ],
    )
    def gather_kernel(idx_hbm, proj_hbm, out_hbm, idx_v, buf0, buf1,
                      sem0, sem1):
        wid = lax.axis_index("s") * NC + lax.axis_index("c")
        base = wid * b_per_w
        pltpu.sync_copy(idx_hbm.at[wid], idx_v)

        def fire(j, buf, sem):
            pltpu.async_copy(proj_hbm.at[idx_v.at[j]], buf, sem)

        def drain_and_out(j, buf, sem):
            pltpu.make_async_copy(proj_hbm.at[idx_v.at[j]], buf, sem).wait()
            pltpu.sync_copy(buf, out_hbm.at[pl.ds(base + j * CH, CH)])

        fire(0, buf0, sem0)

        def body(i, carry):
            j = 2 * i
            fire(j + 1, buf1, sem1)
            drain_and_out(j, buf0, sem0)

            @pl.when(j + 2 < n_chunks)
            def _():
                fire(j + 2, buf0, sem0)

            drain_and_out(j + 1, buf1, sem1)
            return carry

        lax.fori_loop(0, n_chunks // 2, body, 0, unroll=False)

    return gather_kernel


def kernel(input, table, W):
    B = input.shape[0] * input.shape[1]
    rows_pad = ((table.shape[0] + ROW_BLK - 1) // ROW_BLK) * ROW_BLK
    table_pad = jnp.pad(table, ((0, rows_pad - table.shape[0]), (0, 0)))
    proj = _project(table_pad, W)

    idx = input.reshape(NW, B // NW // CH, CH).astype(jnp.int32)
    out = _make_gather(B)(idx, proj)
    return out.reshape(input.shape[0], input.shape[1], EMB)


# speedup vs baseline: 4.7347x; 1.7427x over previous
import functools

import jax
import jax.numpy as jnp
from jax import lax
from jax.experimental import pallas as pl
from jax.experimental.pallas import tpu as pltpu
from jax.experimental.pallas import tpu_sc as plsc

BOTTLENECK = 32
EMB = 128
NC, NS = 2, 16
NW = NC * NS
ROW_BLK = 4096
S = 2


def _proj_body(t_ref, w_ref, o_ref):
    o_ref[...] = jnp.dot(t_ref[...], w_ref[...],
                         preferred_element_type=jnp.float32)


def _project(table, W):
    rows = table.shape[0]
    grid = (rows + ROW_BLK - 1) // ROW_BLK
    return pl.pallas_call(
        _proj_body,
        grid=(grid,),
        in_specs=[
            pl.BlockSpec((ROW_BLK, BOTTLENECK), lambda i: (i, 0)),
            pl.BlockSpec((BOTTLENECK, EMB), lambda i: (0, 0)),
        ],
        out_specs=pl.BlockSpec((ROW_BLK, EMB), lambda i: (i, 0)),
        out_shape=jax.ShapeDtypeStruct((rows, EMB), jnp.float32),
    )(table, W)


def _make_gather(n_batch, n_field):
    slabs_per_w = n_batch // NW
    n_pairs = slabs_per_w // S
    mesh = plsc.VectorSubcoreMesh(core_axis_name="c", subcore_axis_name="s")

    @functools.partial(
        pl.kernel,
        out_type=jax.ShapeDtypeStruct((n_batch, n_field, EMB), jnp.float32),
        mesh=mesh,
        scratch_types=[
            pltpu.VMEM((slabs_per_w, n_field), jnp.int32),
            pltpu.VMEM((S, n_field, EMB), jnp.float32),
            pltpu.VMEM((S, n_field, EMB), jnp.float32),
            pltpu.SemaphoreType.DMA,
            pltpu.SemaphoreType.DMA,
        ],
    )
    def gather_kernel(idx_hbm, proj_hbm, out_hbm, idx_v, buf0, buf1,
                      sem0, sem1):
        wid = lax.axis_index("s") * NC + lax.axis_index("c")
        b0 = wid * slabs_per_w
        pltpu.sync_copy(idx_hbm.at[pl.ds(b0, slabs_per_w)], idx_v)

        def fire(p, buf, sem):
            for t in range(S):
                pltpu.async_copy(proj_hbm.at[idx_v.at[S * p + t]],
                                 buf.at[t], sem)

        def drain_out(p, buf, sem):
            for t in range(S):
                pltpu.make_async_copy(proj_hbm.at[idx_v.at[S * p + t]],
                                      buf.at[t], sem).wait()
            pltpu.sync_copy(buf, out_hbm.at[pl.ds(b0 + S * p, S)])

        fire(0, buf0, sem0)

        def body(i, carry):
            p = 2 * i
            fire(p + 1, buf1, sem1)
            drain_out(p, buf0, sem0)

            @pl.when(p + 2 < n_pairs)
            def _():
                fire(p + 2, buf0, sem0)

            drain_out(p + 1, buf1, sem1)
            return carry

        lax.fori_loop(0, n_pairs // 2, body, 0, unroll=False)

    return gather_kernel


def kernel(input, table, W):
    n_batch, n_field = input.shape
    proj = _project(table, W)
    idx = input.astype(jnp.int32)
    return _make_gather(n_batch, n_field)(idx, proj)
